# baseline (speedup 1.0000x reference)
import functools
import jax
import jax.numpy as jnp
from jax import lax
from jax.experimental import pallas as pl
from jax.experimental.pallas import tpu as pltpu
from jax.experimental.pallas import tpu_sc as plsc

EMBED_DIM = 64
PAIR_DIM = 2 * EMBED_DIM
NUM_CORES = 2
NUM_SUBCORES = 16
NUM_WORKERS = NUM_CORES * NUM_SUBCORES
CHUNK = 512
SEL_ROWS = 16


PACK_COLS = 512


def _tc_pack(weight_t, vocab):
    num_pairs = vocab // 2

    def pack_kernel(wt_ref, out_ref):
        x = wt_ref[...]
        t = x.T.reshape(PACK_COLS // 2, 2, EMBED_DIM)
        out_ref[...] = jnp.concatenate([t[:, 0, :], t[:, 1, :]], axis=1)

    grid = (pl.cdiv(vocab, PACK_COLS),)
    return pl.pallas_call(
        pack_kernel,
        grid=grid,
        in_specs=[pl.BlockSpec((EMBED_DIM, PACK_COLS), lambda i: (0, i))],
        out_specs=pl.BlockSpec((PACK_COLS // 2, PAIR_DIM), lambda i: (i, 0)),
        out_shape=jax.ShapeDtypeStruct((num_pairs, PAIR_DIM), jnp.float32),
    )(weight_t)


def _sc_gather(table2, idx2, num_indices):
    mesh = plsc.VectorSubcoreMesh(core_axis_name="c", subcore_axis_name="s")
    per_worker = num_indices // NUM_WORKERS
    num_chunks = per_worker // CHUNK

    @functools.partial(
        pl.kernel,
        mesh=mesh,
        out_type=jax.ShapeDtypeStruct((num_indices, PAIR_DIM), jnp.float32),
        scratch_types=[
            pltpu.VMEM((CHUNK,), jnp.int32),
            pltpu.VMEM((CHUNK, PAIR_DIM), jnp.float32),
            pltpu.SemaphoreType.DMA,
        ],
    )
    def sc_kernel(table_hbm, idx_hbm, out_hbm, idx_v, rows_v, sem):
        wid = lax.axis_index("s") * NUM_CORES + lax.axis_index("c")
        base = wid * per_worker

        @pl.loop(0, num_chunks)
        def _(c):
            start = base + c * CHUNK
            pltpu.sync_copy(idx_hbm.at[pl.ds(start, CHUNK)], idx_v)
            pltpu.async_copy(table_hbm.at[idx_v], rows_v, sem).wait()
            pltpu.sync_copy(rows_v, out_hbm.at[pl.ds(start, CHUNK)])

    return sc_kernel(table2, idx2)


def _tc_select(pairs, idx_lanes, batch, seq):
    rows_per_step = SEL_ROWS * seq

    lane_rows = rows_per_step // 128

    def sel_kernel(pairs_ref, idx_ref, out_ref):
        pm = (idx_ref[...] & 1).astype(jnp.float32)
        pm3 = pm.reshape(lane_rows, 128, 1)
        pairs3 = pairs_ref[...].reshape(lane_rows, 128, PAIR_DIM)
        sel = (pairs3[:, :, :EMBED_DIM] * (1.0 - pm3)
               + pairs3[:, :, EMBED_DIM:] * pm3)
        out_ref[...] = sel.reshape(SEL_ROWS, seq, EMBED_DIM)

    grid = (batch // SEL_ROWS,)
    return pl.pallas_call(
        sel_kernel,
        grid=grid,
        in_specs=[
            pl.BlockSpec((rows_per_step, PAIR_DIM), lambda i: (i, 0)),
            pl.BlockSpec((1, rows_per_step // 128, 128),
                         lambda i: (i, 0, 0)),
        ],
        out_specs=pl.BlockSpec((SEL_ROWS, seq, EMBED_DIM),
                               lambda i: (i, 0, 0)),
        out_shape=jax.ShapeDtypeStruct((batch, seq, EMBED_DIM),
                                       jnp.float32),
    )(pairs, idx_lanes)


def kernel(batchinput, weight):
    batch, seq = batchinput.shape
    num_indices = batch * seq
    idx_flat = batchinput.reshape(num_indices)
    idx2 = lax.shift_right_logical(idx_flat, 1)
    rows_per_step = SEL_ROWS * seq
    idx_lanes = batchinput.reshape(num_indices // rows_per_step,
                                   rows_per_step // 128, 128)

    table2 = _tc_pack(weight.T, weight.shape[0])
    pairs = _sc_gather(table2, idx2, num_indices)
    return _tc_select(pairs, idx_lanes, batch, seq)


# speedup vs baseline: 1.3244x; 1.3244x over previous
import functools
import jax
import jax.numpy as jnp
from jax import lax
from jax.experimental import pallas as pl
from jax.experimental.pallas import tpu as pltpu
from jax.experimental.pallas import tpu_sc as plsc

EMBED_DIM = 64
PAIR_DIM = 2 * EMBED_DIM
NUM_CORES = 2
NUM_SUBCORES = 16
NUM_WORKERS = NUM_CORES * NUM_SUBCORES
CHUNK = 512
SEL_ROWS = 16


PACK_COLS = 512


def _tc_pack(weight_t, vocab):
    num_blocks = pl.cdiv(vocab, 2 * PACK_COLS)
    num_pairs = num_blocks * PACK_COLS

    def pack_kernel(wt_ref, out_ref):
        out_ref[:, :EMBED_DIM] = wt_ref[:, :PACK_COLS].T
        out_ref[:, EMBED_DIM:] = wt_ref[:, PACK_COLS:].T

    grid = (num_blocks,)
    return pl.pallas_call(
        pack_kernel,
        grid=grid,
        in_specs=[pl.BlockSpec((EMBED_DIM, 2 * PACK_COLS),
                               lambda i: (0, i))],
        out_specs=pl.BlockSpec((PACK_COLS, PAIR_DIM), lambda i: (i, 0)),
        out_shape=jax.ShapeDtypeStruct((num_pairs, PAIR_DIM), jnp.float32),
    )(weight_t)


def _sc_gather(table2, idx2, num_indices):
    mesh = plsc.VectorSubcoreMesh(core_axis_name="c", subcore_axis_name="s")
    per_worker = num_indices // NUM_WORKERS
    num_chunks = per_worker // CHUNK

    @functools.partial(
        pl.kernel,
        mesh=mesh,
        out_type=jax.ShapeDtypeStruct((num_indices, PAIR_DIM), jnp.float32),
        scratch_types=[
            pltpu.VMEM((CHUNK,), jnp.int32),
            pltpu.VMEM((CHUNK, PAIR_DIM), jnp.float32),
            pltpu.SemaphoreType.DMA,
        ],
    )
    def sc_kernel(table_hbm, idx_hbm, out_hbm, idx_v, rows_v, sem):
        wid = lax.axis_index("s") * NUM_CORES + lax.axis_index("c")
        base = wid * per_worker

        @pl.loop(0, num_chunks)
        def _(c):
            start = base + c * CHUNK
            pltpu.sync_copy(idx_hbm.at[pl.ds(start, CHUNK)], idx_v)
            pltpu.async_copy(table_hbm.at[idx_v], rows_v, sem).wait()
            pltpu.sync_copy(rows_v, out_hbm.at[pl.ds(start, CHUNK)])

    return sc_kernel(table2, idx2)


def _tc_select(pairs, idx_lanes, batch, seq):
    rows_per_step = SEL_ROWS * seq

    lane_rows = rows_per_step // 128

    def sel_kernel(pairs_ref, idx_ref, out_ref):
        pm = (lax.shift_right_logical(idx_ref[...], 9) & 1).astype(
            jnp.float32)
        pm3 = pm.reshape(lane_rows, 128, 1)
        pairs3 = pairs_ref[...].reshape(lane_rows, 128, PAIR_DIM)
        sel = (pairs3[:, :, :EMBED_DIM] * (1.0 - pm3)
               + pairs3[:, :, EMBED_DIM:] * pm3)
        out_ref[...] = sel.reshape(SEL_ROWS, seq, EMBED_DIM)

    grid = (batch // SEL_ROWS,)
    return pl.pallas_call(
        sel_kernel,
        grid=grid,
        in_specs=[
            pl.BlockSpec((rows_per_step, PAIR_DIM), lambda i: (i, 0)),
            pl.BlockSpec((1, rows_per_step // 128, 128),
                         lambda i: (i, 0, 0)),
        ],
        out_specs=pl.BlockSpec((SEL_ROWS, seq, EMBED_DIM),
                               lambda i: (i, 0, 0)),
        out_shape=jax.ShapeDtypeStruct((batch, seq, EMBED_DIM),
                                       jnp.float32),
    )(pairs, idx_lanes)


def kernel(batchinput, weight):
    batch, seq = batchinput.shape
    num_indices = batch * seq
    idx_flat = batchinput.reshape(num_indices)
    idx2 = (lax.shift_left(lax.shift_right_logical(idx_flat, 10), 9)
            | (idx_flat & 511))
    rows_per_step = SEL_ROWS * seq
    idx_lanes = batchinput.reshape(num_indices // rows_per_step,
                                   rows_per_step // 128, 128)

    table2 = _tc_pack(weight.T, weight.shape[0])
    pairs = _sc_gather(table2, idx2, num_indices)
    return _tc_select(pairs, idx_lanes, batch, seq)


# speedup vs baseline: 1.6719x; 1.2623x over previous
import functools
import jax
import jax.numpy as jnp
from jax import lax
from jax.experimental import pallas as pl
from jax.experimental.pallas import tpu as pltpu
from jax.experimental.pallas import tpu_sc as plsc

EMBED_DIM = 64
PAIR_DIM = 2 * EMBED_DIM
NUM_CORES = 2
NUM_SUBCORES = 16
NUM_WORKERS = NUM_CORES * NUM_SUBCORES
CHUNK = 512
SEL_ROWS = 16


PACK_COLS = 2048
PACK_SHIFT = PACK_COLS.bit_length() - 1


def _tc_pack(weight_t, vocab):
    num_blocks = pl.cdiv(vocab, 2 * PACK_COLS)
    num_pairs = num_blocks * PACK_COLS

    def pack_kernel(wt_ref, out_ref):
        out_ref[:, :EMBED_DIM] = wt_ref[:, :PACK_COLS].T
        out_ref[:, EMBED_DIM:] = wt_ref[:, PACK_COLS:].T

    grid = (num_blocks,)
    return pl.pallas_call(
        pack_kernel,
        grid=grid,
        in_specs=[pl.BlockSpec((EMBED_DIM, 2 * PACK_COLS),
                               lambda i: (0, i))],
        out_specs=pl.BlockSpec((PACK_COLS, PAIR_DIM), lambda i: (i, 0)),
        out_shape=jax.ShapeDtypeStruct((num_pairs, PAIR_DIM), jnp.float32),
    )(weight_t)


def _sc_gather(table2, idx2, num_indices):
    mesh = plsc.VectorSubcoreMesh(core_axis_name="c", subcore_axis_name="s")
    per_worker = num_indices // NUM_WORKERS
    num_chunks = per_worker // CHUNK

    @functools.partial(
        pl.kernel,
        mesh=mesh,
        out_type=jax.ShapeDtypeStruct((num_indices, PAIR_DIM), jnp.float32),
        scratch_types=[
            pltpu.VMEM((CHUNK,), jnp.int32),
            pltpu.VMEM((CHUNK, PAIR_DIM), jnp.float32),
            pltpu.SemaphoreType.DMA,
        ],
    )
    def sc_kernel(table_hbm, idx_hbm, out_hbm, idx_v, rows_v, sem):
        wid = lax.axis_index("s") * NUM_CORES + lax.axis_index("c")
        base = wid * per_worker

        @pl.loop(0, num_chunks)
        def _(c):
            start = base + c * CHUNK
            pltpu.sync_copy(idx_hbm.at[pl.ds(start, CHUNK)], idx_v)
            pltpu.async_copy(table_hbm.at[idx_v], rows_v, sem).wait()
            pltpu.sync_copy(rows_v, out_hbm.at[pl.ds(start, CHUNK)])

    return sc_kernel(table2, idx2)


def _tc_select(pairs, idx_lanes, batch, seq):
    rows_per_step = SEL_ROWS * seq

    lane_rows = rows_per_step // 128

    def sel_kernel(pairs_ref, idx_ref, out_ref):
        pm = (lax.shift_right_logical(idx_ref[...], PACK_SHIFT) & 1).astype(
            jnp.float32)
        pm3 = pm.reshape(lane_rows, 128, 1)
        pairs3 = pairs_ref[...].reshape(lane_rows, 128, PAIR_DIM)
        sel = (pairs3[:, :, :EMBED_DIM] * (1.0 - pm3)
               + pairs3[:, :, EMBED_DIM:] * pm3)
        out_ref[...] = sel.reshape(SEL_ROWS, seq, EMBED_DIM)

    grid = (batch // SEL_ROWS,)
    return pl.pallas_call(
        sel_kernel,
        grid=grid,
        in_specs=[
            pl.BlockSpec((rows_per_step, PAIR_DIM), lambda i: (i, 0)),
            pl.BlockSpec((1, rows_per_step // 128, 128),
                         lambda i: (i, 0, 0)),
        ],
        out_specs=pl.BlockSpec((SEL_ROWS, seq, EMBED_DIM),
                               lambda i: (i, 0, 0)),
        out_shape=jax.ShapeDtypeStruct((batch, seq, EMBED_DIM),
                                       jnp.float32),
    )(pairs, idx_lanes)


def kernel(batchinput, weight):
    batch, seq = batchinput.shape
    num_indices = batch * seq
    idx_flat = batchinput.reshape(num_indices)
    idx2 = (lax.shift_left(lax.shift_right_logical(idx_flat, PACK_SHIFT + 1),
                           PACK_SHIFT)
            | (idx_flat & (PACK_COLS - 1)))
    rows_per_step = SEL_ROWS * seq
    idx_lanes = batchinput.reshape(num_indices // rows_per_step,
                                   rows_per_step // 128, 128)

    table2 = _tc_pack(weight.T, weight.shape[0])
    pairs = _sc_gather(table2, idx2, num_indices)
    return _tc_select(pairs, idx_lanes, batch, seq)


# speedup vs baseline: 1.8145x; 1.0853x over previous
import functools
import jax
import jax.numpy as jnp
from jax import lax
from jax.experimental import pallas as pl
from jax.experimental.pallas import tpu as pltpu
from jax.experimental.pallas import tpu_sc as plsc

EMBED_DIM = 64
PAIR_DIM = 2 * EMBED_DIM
NUM_CORES = 2
NUM_SUBCORES = 16
NUM_WORKERS = NUM_CORES * NUM_SUBCORES
CHUNK = 512
SEL_ROWS = 16


PACK_COLS = 2048
PACK_SHIFT = PACK_COLS.bit_length() - 1


def _tc_pack(weight_t, vocab):
    num_blocks = pl.cdiv(vocab, 2 * PACK_COLS)
    num_pairs = num_blocks * PACK_COLS

    def pack_kernel(wt_ref, out_ref):
        out_ref[:, :EMBED_DIM] = wt_ref[:, :PACK_COLS].T
        out_ref[:, EMBED_DIM:] = wt_ref[:, PACK_COLS:].T

    grid = (num_blocks,)
    return pl.pallas_call(
        pack_kernel,
        grid=grid,
        in_specs=[pl.BlockSpec((EMBED_DIM, 2 * PACK_COLS),
                               lambda i: (0, i))],
        out_specs=pl.BlockSpec((PACK_COLS, PAIR_DIM), lambda i: (i, 0)),
        out_shape=jax.ShapeDtypeStruct((num_pairs, PAIR_DIM), jnp.float32),
    )(weight_t)


def _sc_gather(table2, idx2, num_indices):
    mesh = plsc.VectorSubcoreMesh(core_axis_name="c", subcore_axis_name="s")
    per_worker = num_indices // NUM_WORKERS
    num_chunks = per_worker // CHUNK

    @functools.partial(
        pl.kernel,
        mesh=mesh,
        out_type=jax.ShapeDtypeStruct((num_indices, PAIR_DIM), jnp.float32),
        scratch_types=[
            pltpu.VMEM((CHUNK,), jnp.int32),
            pltpu.VMEM((CHUNK, PAIR_DIM), jnp.float32),
            pltpu.SemaphoreType.DMA,
        ],
    )
    def sc_kernel(table_hbm, idx_hbm, out_hbm, idx_v, rows_v, sem):
        wid = lax.axis_index("s") * NUM_CORES + lax.axis_index("c")
        base = wid * per_worker

        @pl.loop(0, num_chunks)
        def _(c):
            start = base + c * CHUNK
            pltpu.sync_copy(idx_hbm.at[pl.ds(start, CHUNK)], idx_v)
            pltpu.async_copy(table_hbm.at[idx_v], rows_v, sem).wait()
            pltpu.sync_copy(rows_v, out_hbm.at[pl.ds(start, CHUNK)])

    return sc_kernel(table2, idx2)


def _tc_select(pairs, idx_lanes, batch, seq):
    rows_per_step = SEL_ROWS * seq

    lane_rows = rows_per_step // 128

    def sel_kernel(pairs_ref, idx_ref, out_ref):
        pm = (lax.shift_right_logical(idx_ref[...], PACK_SHIFT) & 1).astype(
            jnp.float32)
        pm3 = pm.reshape(lane_rows, 128, 1)
        pairs3 = pairs_ref[...].reshape(lane_rows, 128, PAIR_DIM)
        sel = (pairs3[:, :, :EMBED_DIM] * (1.0 - pm3)
               + pairs3[:, :, EMBED_DIM:] * pm3)
        out_ref[...] = sel.reshape(rows_per_step, EMBED_DIM)

    grid = (batch // SEL_ROWS,)
    return pl.pallas_call(
        sel_kernel,
        grid=grid,
        in_specs=[
            pl.BlockSpec((rows_per_step, PAIR_DIM), lambda i: (i, 0)),
            pl.BlockSpec((1, rows_per_step // 128, 128),
                         lambda i: (i, 0, 0)),
        ],
        out_specs=pl.BlockSpec((rows_per_step, EMBED_DIM),
                               lambda i: (i, 0)),
        out_shape=jax.ShapeDtypeStruct((batch * seq, EMBED_DIM),
                                       jnp.float32),
    )(pairs, idx_lanes)


def kernel(batchinput, weight):
    batch, seq = batchinput.shape
    num_indices = batch * seq
    idx_flat = batchinput.reshape(num_indices)
    idx2 = (lax.shift_left(lax.shift_right_logical(idx_flat, PACK_SHIFT + 1),
                           PACK_SHIFT)
            | (idx_flat & (PACK_COLS - 1)))
    rows_per_step = SEL_ROWS * seq
    idx_lanes = batchinput.reshape(num_indices // rows_per_step,
                                   rows_per_step // 128, 128)

    table2 = _tc_pack(weight.T, weight.shape[0])
    pairs = _sc_gather(table2, idx2, num_indices)
    out = _tc_select(pairs, idx_lanes, batch, seq)
    return out.reshape(batch, seq, EMBED_DIM)
